# baseline (speedup 1.0000x reference)
import functools

import jax
import jax.numpy as jnp
from jax import lax
from jax.experimental import pallas as pl
from jax.experimental.pallas import tpu as pltpu
from jax.experimental.pallas import tpu_sc as plsc

_N = 10000
_D = 128
_NW = 32
_EPW = 10240
_EPAD = _EPW * _NW
_ROWS = 10240
_RPT = _ROWS // 16


def _sc_segsum(with_counts, ch, gc):
    nchunk = _EPW // ch
    mesh = plsc.VectorSubcoreMesh(core_axis_name="c", subcore_axis_name="s")
    out_type = [jax.ShapeDtypeStruct((2 * _ROWS, _D), jnp.float32)]
    nbuf = 4
    scratch = [
        pltpu.VMEM((gc, ch), jnp.int32),
        pltpu.VMEM((gc, ch), jnp.int32),
        pltpu.VMEM((nbuf, ch, _D), jnp.float32),
        pltpu.VMEM_SHARED((_ROWS, _D), jnp.float32),
    ] + [pltpu.SemaphoreType.DMA] * (2 * nbuf)
    if with_counts:
        out_type.append(jax.ShapeDtypeStruct((_NW * _ROWS,), jnp.float32))
        scratch.append(pltpu.VMEM((_ROWS,), jnp.float32))

    cparams = pltpu.CompilerParams(needs_layout_passes=not with_counts)

    @functools.partial(pl.kernel, mesh=mesh, out_type=out_type,
                       scratch_types=scratch, compiler_params=cparams)
    def body(table_hbm, src_hbm, dst_hbm, zrows_hbm, *rest):
        if with_counts:
            (zcnt_hbm, sums_out, cnts_out, src_v, dst_v, rows_v, acc_sh,
             *sems, hist_v) = rest
        else:
            (sums_out, src_v, dst_v, rows_v, acc_sh, *sems) = rest
        gs = sems[:nbuf]
        ss = sems[nbuf:]
        c = lax.axis_index("c")
        s = lax.axis_index("s")
        wid = c * 16 + s
        base = s * _RPT

        pltpu.sync_copy(zrows_hbm, acc_sh.at[pl.ds(base, _RPT)])
        if with_counts:
            pltpu.sync_copy(zcnt_hbm, hist_v)
        plsc.subcore_barrier()

        def gath(j, b):
            return pltpu.async_copy(table_hbm.at[src_v.at[j]], rows_v.at[b],
                                    gs[b])

        def scat(j, b):
            return pltpu.async_copy(rows_v.at[b], acc_sh.at[dst_v.at[j]],
                                    ss[b], add=True)

        if with_counts:
            ones16 = jnp.full((16,), 1.0, jnp.float32)

            def hist(j):
                for q in range(ch // 16):
                    idxv = dst_v[j, pl.ds(q * 16, 16)]
                    plsc.addupdate_scatter(hist_v, [idxv], ones16)

        @pl.loop(0, nchunk // gc)
        def group(g):
            row0 = wid * nchunk + g * gc
            pltpu.sync_copy(src_hbm.at[pl.ds(row0, gc)], src_v)
            pltpu.sync_copy(dst_hbm.at[pl.ds(row0, gc)], dst_v)

            @pl.loop(0, gc // (2 * nbuf))
            def step(j):
                j0 = j * 2 * nbuf
                g = [gath(j0 + k, k) for k in range(nbuf)]
                if with_counts:
                    for k in range(2 * nbuf):
                        hist(j0 + k)
                sc = [None] * nbuf
                for k in range(nbuf):
                    g[k].wait()
                    sc[k] = scat(j0 + k, k)
                for k in range(nbuf):
                    sc[k].wait()
                    g[k] = gath(j0 + nbuf + k, k)
                for k in range(nbuf):
                    g[k].wait()
                    sc[k] = scat(j0 + nbuf + k, k)
                for k in range(nbuf):
                    sc[k].wait()

        plsc.subcore_barrier()

        orow = c * _ROWS + base
        pltpu.sync_copy(acc_sh.at[pl.ds(base, _RPT)],
                        sums_out.at[pl.ds(orow, _RPT)])
        if with_counts:
            pltpu.sync_copy(hist_v, cnts_out.at[pl.ds(wid * _ROWS, _ROWS)])

    return body


_CH = 32
_GC = 40
_sc_segsum_counts = _sc_segsum(True, _CH, _GC)
_sc_segsum_plain = _sc_segsum(False, _CH, _GC)


def _tc_layer(relu):
    R = 2048
    G = (_N + R - 1) // R

    def body(s_ref, c_ref, h_ref, wl_ref, wr_ref, b_ref, o_ref):
        ssum = s_ref[0] + s_ref[1]
        ones_nw = jnp.ones((_NW, 1), jnp.float32)
        deg = lax.dot_general(c_ref[...], ones_nw, (((0,), (0,)), ((), ())),
                              preferred_element_type=jnp.float32)
        agg = ssum * (1.0 / jnp.maximum(deg, 1.0))
        dn = (((1,), (1,)), ((), ()))
        acc = lax.dot_general(agg, wl_ref[...], dn,
                              preferred_element_type=jnp.float32)
        acc = acc + lax.dot_general(h_ref[...], wr_ref[...], dn,
                                    preferred_element_type=jnp.float32)
        acc = acc + b_ref[...]
        if relu:
            acc = jnp.maximum(acc, 0.0)
        o_ref[...] = acc

    return pl.pallas_call(
        body,
        grid=(G,),
        in_specs=[
            pl.BlockSpec((2, R, _D), lambda i: (0, i, 0)),
            pl.BlockSpec((_NW, R), lambda i: (0, i)),
            pl.BlockSpec((R, _D), lambda i: (i, 0)),
            pl.BlockSpec((_D, _D), lambda i: (0, 0)),
            pl.BlockSpec((_D, _D), lambda i: (0, 0)),
            pl.BlockSpec((1, _D), lambda i: (0, 0)),
        ],
        out_specs=pl.BlockSpec((R, _D), lambda i: (i, 0)),
        out_shape=jax.ShapeDtypeStruct((_N, _D), jnp.float32),
    )


_tc_layer_relu = _tc_layer(True)
_tc_layer_plain = _tc_layer(False)


def kernel(x, edge_index, W1_l, W1_r, b1, W2_l, W2_r, b2):
    src = edge_index[0].astype(jnp.int32)
    dst = edge_index[1].astype(jnp.int32)
    npad = _EPAD - src.shape[0]
    fill = jnp.arange(npad, dtype=jnp.int32)
    src_p = jnp.concatenate([src, fill % jnp.int32(_N)])
    dst_p = jnp.concatenate([dst, jnp.int32(_N) + fill % jnp.int32(_ROWS - _N)])
    nchunk = _EPW // _CH
    src2 = src_p.reshape(_NW * nchunk, _CH)
    dst2 = dst_p.reshape(_NW * nchunk, _CH)
    zrows = jnp.zeros((_RPT, _D), jnp.float32)
    zcnt = jnp.zeros((_ROWS,), jnp.float32)

    sums1, cnts = _sc_segsum_counts(x, src2, dst2, zrows, zcnt)
    s1 = sums1.reshape(2, _ROWS, _D)
    c1 = cnts.reshape(_NW, _ROWS)
    h1 = _tc_layer_relu(s1, c1, x, W1_l, W1_r, b1.reshape(1, _D))

    sums2 = _sc_segsum_plain(h1, src2, dst2, zrows)
    if isinstance(sums2, (list, tuple)):
        sums2 = sums2[0]
    s2 = sums2.reshape(2, _ROWS, _D)
    h2 = _tc_layer_plain(s2, c1, h1, W2_l, W2_r, b2.reshape(1, _D))
    return h2


# speedup vs baseline: 1.0252x; 1.0252x over previous
import functools

import jax
import jax.numpy as jnp
from jax import lax
from jax.experimental import pallas as pl
from jax.experimental.pallas import tpu as pltpu
from jax.experimental.pallas import tpu_sc as plsc

_N = 10000
_D = 128
_NW = 32
_EPW = 10752
_EPAD = _EPW * _NW
_ROWS = 10240
_RPT = _ROWS // 16


def _sc_segsum(with_counts, ch, gc):
    nchunk = _EPW // ch
    mesh = plsc.VectorSubcoreMesh(core_axis_name="c", subcore_axis_name="s")
    out_type = [jax.ShapeDtypeStruct((2 * _ROWS, _D), jnp.float32)]
    nbuf = 4 if with_counts else 6
    scratch = [
        pltpu.VMEM((gc, ch), jnp.int32),
        pltpu.VMEM((gc, ch), jnp.int32),
        pltpu.VMEM((nbuf, ch, _D), jnp.float32),
        pltpu.VMEM_SHARED((_ROWS, _D), jnp.float32),
    ] + [pltpu.SemaphoreType.DMA] * (2 * nbuf)
    if with_counts:
        out_type.append(jax.ShapeDtypeStruct((_NW * _ROWS,), jnp.float32))
        scratch.append(pltpu.VMEM((_ROWS,), jnp.float32))

    cparams = pltpu.CompilerParams(needs_layout_passes=not with_counts)

    @functools.partial(pl.kernel, mesh=mesh, out_type=out_type,
                       scratch_types=scratch, compiler_params=cparams)
    def body(table_hbm, src_hbm, dst_hbm, zrows_hbm, *rest):
        if with_counts:
            (zcnt_hbm, sums_out, cnts_out, src_v, dst_v, rows_v, acc_sh,
             *sems, hist_v) = rest
        else:
            (sums_out, src_v, dst_v, rows_v, acc_sh, *sems) = rest
        gs = sems[:nbuf]
        ss = sems[nbuf:]
        c = lax.axis_index("c")
        s = lax.axis_index("s")
        wid = c * 16 + s
        base = s * _RPT

        pltpu.sync_copy(zrows_hbm, acc_sh.at[pl.ds(base, _RPT)])
        if with_counts:
            pltpu.sync_copy(zcnt_hbm, hist_v)
        plsc.subcore_barrier()

        def gath(j, b):
            return pltpu.async_copy(table_hbm.at[src_v.at[j]], rows_v.at[b],
                                    gs[b])

        def scat(j, b):
            return pltpu.async_copy(rows_v.at[b], acc_sh.at[dst_v.at[j]],
                                    ss[b], add=True)

        if with_counts:
            ones16 = jnp.full((16,), 1.0, jnp.float32)

            def hist(j):
                for q in range(ch // 16):
                    idxv = dst_v[j, pl.ds(q * 16, 16)]
                    plsc.addupdate_scatter(hist_v, [idxv], ones16)

        @pl.loop(0, nchunk // gc)
        def group(g):
            row0 = wid * nchunk + g * gc
            pltpu.sync_copy(src_hbm.at[pl.ds(row0, gc)], src_v)
            pltpu.sync_copy(dst_hbm.at[pl.ds(row0, gc)], dst_v)

            @pl.loop(0, gc // (2 * nbuf))
            def step(j):
                j0 = j * 2 * nbuf
                g = [gath(j0 + k, k) for k in range(nbuf)]
                if with_counts:
                    for k in range(2 * nbuf):
                        hist(j0 + k)
                sc = [None] * nbuf
                for k in range(nbuf):
                    g[k].wait()
                    sc[k] = scat(j0 + k, k)
                for k in range(nbuf):
                    sc[k].wait()
                    g[k] = gath(j0 + nbuf + k, k)
                for k in range(nbuf):
                    g[k].wait()
                    sc[k] = scat(j0 + nbuf + k, k)
                for k in range(nbuf):
                    sc[k].wait()

        plsc.subcore_barrier()

        orow = c * _ROWS + base
        pltpu.sync_copy(acc_sh.at[pl.ds(base, _RPT)],
                        sums_out.at[pl.ds(orow, _RPT)])
        if with_counts:
            pltpu.sync_copy(hist_v, cnts_out.at[pl.ds(wid * _ROWS, _ROWS)])

    return body


_CH = 32
_GC = 48
_sc_segsum_counts = _sc_segsum(True, _CH, _GC)
_sc_segsum_plain = _sc_segsum(False, _CH, _GC)


def _tc_layer(relu):
    R = 2048
    G = (_N + R - 1) // R

    def body(s_ref, c_ref, h_ref, wl_ref, wr_ref, b_ref, o_ref):
        ssum = s_ref[0] + s_ref[1]
        ones_nw = jnp.ones((_NW, 1), jnp.float32)
        deg = lax.dot_general(c_ref[...], ones_nw, (((0,), (0,)), ((), ())),
                              preferred_element_type=jnp.float32)
        agg = ssum * (1.0 / jnp.maximum(deg, 1.0))
        dn = (((1,), (1,)), ((), ()))
        acc = lax.dot_general(agg, wl_ref[...], dn,
                              preferred_element_type=jnp.float32)
        acc = acc + lax.dot_general(h_ref[...], wr_ref[...], dn,
                                    preferred_element_type=jnp.float32)
        acc = acc + b_ref[...]
        if relu:
            acc = jnp.maximum(acc, 0.0)
        o_ref[...] = acc

    return pl.pallas_call(
        body,
        grid=(G,),
        in_specs=[
            pl.BlockSpec((2, R, _D), lambda i: (0, i, 0)),
            pl.BlockSpec((_NW, R), lambda i: (0, i)),
            pl.BlockSpec((R, _D), lambda i: (i, 0)),
            pl.BlockSpec((_D, _D), lambda i: (0, 0)),
            pl.BlockSpec((_D, _D), lambda i: (0, 0)),
            pl.BlockSpec((1, _D), lambda i: (0, 0)),
        ],
        out_specs=pl.BlockSpec((R, _D), lambda i: (i, 0)),
        out_shape=jax.ShapeDtypeStruct((_N, _D), jnp.float32),
    )


_tc_layer_relu = _tc_layer(True)
_tc_layer_plain = _tc_layer(False)


def kernel(x, edge_index, W1_l, W1_r, b1, W2_l, W2_r, b2):
    src = edge_index[0].astype(jnp.int32)
    dst = edge_index[1].astype(jnp.int32)
    npad = _EPAD - src.shape[0]
    fill = jnp.arange(npad, dtype=jnp.int32)
    src_p = jnp.concatenate([src, fill % jnp.int32(_N)])
    dst_p = jnp.concatenate([dst, jnp.int32(_N) + fill % jnp.int32(_ROWS - _N)])
    nchunk = _EPW // _CH
    src2 = src_p.reshape(_NW * nchunk, _CH)
    dst2 = dst_p.reshape(_NW * nchunk, _CH)
    zrows = jnp.zeros((_RPT, _D), jnp.float32)
    zcnt = jnp.zeros((_ROWS,), jnp.float32)

    sums1, cnts = _sc_segsum_counts(x, src2, dst2, zrows, zcnt)
    s1 = sums1.reshape(2, _ROWS, _D)
    c1 = cnts.reshape(_NW, _ROWS)
    h1 = _tc_layer_relu(s1, c1, x, W1_l, W1_r, b1.reshape(1, _D))

    sums2 = _sc_segsum_plain(h1, src2, dst2, zrows)
    if isinstance(sums2, (list, tuple)):
        sums2 = sums2[0]
    s2 = sums2.reshape(2, _ROWS, _D)
    h2 = _tc_layer_plain(s2, c1, h1, W2_l, W2_r, b2.reshape(1, _D))
    return h2
